# baseline (speedup 1.0000x reference)
import jax
import jax.numpy as jnp
from jax import lax
from jax.experimental import pallas as pl
from jax.experimental.pallas import tpu as pltpu
from jax.experimental.pallas import tpu_sc as plsc

_B, _C, _H, _W = 16, 192, 64, 64
_BB = 2


def _sc_gather_body(n_hbm, gm_hbm, out_hbm, n_v, idx_v, rows_v, sem):
    wid = lax.axis_index("s") * 2 + lax.axis_index("c")

    @pl.when(wid < 2)
    def _():
        pltpu.sync_copy(n_hbm, n_v)
        nv = n_v[...]
        idx_v[...] = (nv - lax.rem(nv, 1.0)).astype(jnp.int32) + wid
        pltpu.async_copy(gm_hbm.at[idx_v], rows_v, sem).wait()
        pltpu.sync_copy(rows_v, out_hbm.at[wid])


_sc_gather = pl.kernel(
    _sc_gather_body,
    out_type=jax.ShapeDtypeStruct((2, _B, _C), jnp.float32),
    mesh=plsc.VectorSubcoreMesh(core_axis_name="c", subcore_axis_name="s"),
    compiler_params=pltpu.CompilerParams(use_tc_tiling_on_sc=False),
    scratch_types=[
        pltpu.VMEM((_B,), jnp.float32),
        pltpu.VMEM((_B,), jnp.int32),
        pltpu.VMEM((_B, _C), jnp.float32),
        pltpu.SemaphoreType.DMA,
    ],
)


def _gain_scale_body(n_ref, g12_ref, x_ref, o_ref):
    i = pl.program_id(0)
    gains = []
    for j in range(_BB):
        idx = _BB * i + j
        nb = n_ref[idx]
        l = nb - jnp.floor(nb)
        g1 = jnp.abs(g12_ref[0, pl.ds(idx, 1), :])
        g2 = jnp.abs(g12_ref[1, pl.ds(idx, 1), :])
        gains.append(g1 ** (1.0 - l) * g2 ** l)
    gain = jnp.concatenate(gains, axis=0)
    o_ref[...] = x_ref[...] * gain.reshape(_BB, 1, 1, _C)


def kernel(x, n, gain_matrix):
    g12 = _sc_gather(n, gain_matrix)
    xt = jnp.transpose(x, (0, 2, 3, 1))
    out = pl.pallas_call(
        _gain_scale_body,
        grid=(_B // _BB,),
        in_specs=[
            pl.BlockSpec(memory_space=pltpu.SMEM),
            pl.BlockSpec((2, _B, _C), lambda i: (0, 0, 0)),
            pl.BlockSpec((_BB, _H, _W, _C), lambda i: (i, 0, 0, 0)),
        ],
        out_specs=pl.BlockSpec((_BB, _H, _W, _C), lambda i: (i, 0, 0, 0)),
        out_shape=jax.ShapeDtypeStruct((_B, _H, _W, _C), jnp.float32),
    )(n, g12, xt)
    return jnp.transpose(out, (0, 3, 1, 2))


# speedup vs baseline: 1.5356x; 1.5356x over previous
import jax
import jax.numpy as jnp
from jax.experimental import pallas as pl
from jax.experimental.pallas import tpu as pltpu

_B, _C, _H, _W = 16, 192, 64, 64
_BB = 2


def _gain_scale_body(n_ref, gm_ref, x_ref, o_ref):
    i = pl.program_id(0)
    gains = []
    for j in range(_BB):
        nb = n_ref[_BB * i + j]
        nf = jnp.floor(nb)
        l = nb - nf
        ni = nf.astype(jnp.int32)
        g1 = jnp.abs(gm_ref[pl.ds(ni, 1), :])
        g2 = jnp.abs(gm_ref[pl.ds(ni + 1, 1), :])
        gains.append(g1 ** (1.0 - l) * g2 ** l)
    gain = jnp.concatenate(gains, axis=0)
    o_ref[...] = x_ref[...] * gain.reshape(_BB, 1, 1, _C)


def kernel(x, n, gain_matrix):
    xt = jnp.transpose(x, (0, 2, 3, 1))
    out = pl.pallas_call(
        _gain_scale_body,
        grid=(_B // _BB,),
        in_specs=[
            pl.BlockSpec(memory_space=pltpu.SMEM),
            pl.BlockSpec((6, _C), lambda i: (0, 0)),
            pl.BlockSpec((_BB, _H, _W, _C), lambda i: (i, 0, 0, 0)),
        ],
        out_specs=pl.BlockSpec((_BB, _H, _W, _C), lambda i: (i, 0, 0, 0)),
        out_shape=jax.ShapeDtypeStruct((_B, _H, _W, _C), jnp.float32),
    )(n, gain_matrix, xt)
    return jnp.transpose(out, (0, 3, 1, 2))
